# baseline (speedup 1.0000x reference)
import functools

import jax
import jax.numpy as jnp
import numpy as np
from jax import lax
from jax.experimental import pallas as pl
from jax.experimental.pallas import tpu as pltpu
from jax.experimental.pallas import tpu_sc as plsc

_NC = 2
_NS = 16
_NW = _NC * _NS
_L = 16

_W = 128
_PIECE = 16


@functools.lru_cache(maxsize=None)
def _build_sc_gather(vocab: int, embed: int, batch: int, tp: int):
    assert embed == _L and tp == 8
    n_chunks = batch // tp
    bw = batch // _NW
    wpi = n_chunks // bw
    xc = bw * tp
    n_p = bw // _PIECE

    mesh = plsc.VectorSubcoreMesh(core_axis_name="c", subcore_axis_name="s")

    @functools.partial(
        pl.kernel,
        out_type=jax.ShapeDtypeStruct((tp, n_chunks * embed), jnp.float32),
        mesh=mesh,
        scratch_types=[
            pltpu.VMEM((xc + _L,), jnp.int32),
            pltpu.SMEM((bw,), jnp.int32),
            pltpu.VMEM((3, _PIECE, embed, _W), jnp.float32),
            pltpu.VMEM((bw * embed,), jnp.float32),
            pltpu.SemaphoreType.DMA,
            pltpu.SemaphoreType.DMA,
            pltpu.SemaphoreType.DMA,
        ],
        compiler_params=pltpu.CompilerParams(
            needs_layout_passes=False, skip_device_barrier=True
        ),
    )
    def k(x_hbm, tblT_hbm, out_hbm, xraw_v, base_s, win_v, rv_v, s0, s1, s2):
        sems = (s0, s1, s2)
        wid = lax.axis_index("s") * _NC + lax.axis_index("c")
        i = wid // wpi
        j0 = (wid % wpi) * bw
        lanes = lax.iota(jnp.int32, _L)
        zero = lanes * 0
        m_lo = lanes == i
        m_hi = lanes == i + tp
        pltpu.sync_copy(x_hbm.at[pl.ds(j0 * tp, xc)], xraw_v.at[pl.ds(0, xc)])

        for u2 in range(bw // 2):
            xw = xraw_v[pl.ds(2 * u2 * tp, _L)]
            for s, m in ((2 * u2, m_lo), (2 * u2 + 1, m_hi)):
                xv = jnp.sum(jnp.where(m, xw, zero), dtype=jnp.int32)
                base_s[s] = xv & ~(_W - 1)

        def fire(p, buf, sem):
            for u in range(_PIECE):
                a = pl.multiple_of(base_s[p * _PIECE + u], _W)
                pltpu.async_copy(
                    tblT_hbm.at[:, pl.ds(a, _W)],
                    win_v.at[np.int32(buf), np.int32(u)],
                    sem,
                )

        def drain(buf, sem):
            for u in range(_PIECE):
                pltpu.make_async_copy(
                    tblT_hbm.at[:, pl.ds(0, _W)],
                    win_v.at[np.int32(buf), np.int32(u)],
                    sem,
                ).wait()

        def extract(p, buf):
            bufv = zero + buf
            jb = p * _PIECE
            xvv = plsc.load_gather(xraw_v, [(lanes + jb) * tp + i])
            col = xvv & (_W - 1)
            base = (lanes + jb) * embed
            for e in range(embed):
                vals = plsc.load_gather(win_v, [bufv, lanes, zero + e, col])
                plsc.store_scatter(rv_v, [base + e], vals)

        for r in range(3):
            fire(r, r, sems[r])

        n_main = (n_p - 2) // 3

        def body(q, carry):
            for r in range(3):
                p = 3 * q + r
                drain(r, sems[r])
                extract(p, r)

                @pl.when(p + 3 < n_p)
                def _():
                    fire(p + 3, r, sems[r])

            return carry

        lax.fori_loop(
            jnp.int32(0), jnp.int32(n_main), body, jnp.int32(0), unroll=False
        )
        for p in range(3 * n_main, n_p):
            r = p % 3
            drain(r, sems[r])
            extract(p, r)

        pltpu.sync_copy(rv_v, out_hbm.at[i, pl.ds(j0 * embed, bw * embed)])

    return k


def kernel(x, table):
    vocab, embed = table.shape
    (batch,) = x.shape
    tp = 8
    xi = x.astype(jnp.int32)
    return _build_sc_gather(vocab, embed, batch, tp)(
        xi, table.astype(jnp.float32).T
    )


# speedup vs baseline: 1.0383x; 1.0383x over previous
import functools

import jax
import jax.numpy as jnp
import numpy as np
from jax import lax
from jax.experimental import pallas as pl
from jax.experimental.pallas import tpu as pltpu
from jax.experimental.pallas import tpu_sc as plsc

_NC = 2
_NS = 16
_NW = _NC * _NS
_L = 16

_W = 128
_PIECE = 16


@functools.lru_cache(maxsize=None)
def _build_sc_gather(vocab: int, embed: int, batch: int, tp: int):
    assert embed == _L and tp == 8
    n_chunks = batch // tp
    bw = batch // _NW
    wpi = n_chunks // bw
    xc = bw * tp
    n_p = bw // _PIECE

    mesh = plsc.VectorSubcoreMesh(core_axis_name="c", subcore_axis_name="s")

    @functools.partial(
        pl.kernel,
        out_type=jax.ShapeDtypeStruct((tp, n_chunks * embed), jnp.float32),
        mesh=mesh,
        scratch_types=[
            pltpu.VMEM((xc + _L,), jnp.int32),
            pltpu.SMEM((bw,), jnp.int32),
            pltpu.VMEM((3, embed, _PIECE * _W), jnp.float32),
            pltpu.VMEM((bw * embed,), jnp.float32),
            pltpu.SemaphoreType.DMA,
            pltpu.SemaphoreType.DMA,
            pltpu.SemaphoreType.DMA,
        ],
        compiler_params=pltpu.CompilerParams(
            needs_layout_passes=False, skip_device_barrier=True
        ),
    )
    def k(x_hbm, tblT_hbm, out_hbm, xraw_v, base_s, win_v, rv_v, s0, s1, s2):
        sems = (s0, s1, s2)
        wid = lax.axis_index("s") * _NC + lax.axis_index("c")
        i = wid // wpi
        j0 = (wid % wpi) * bw
        lanes = lax.iota(jnp.int32, _L)
        zero = lanes * 0
        m_lo = lanes == i
        m_hi = lanes == i + tp
        pltpu.sync_copy(x_hbm.at[pl.ds(j0 * tp, xc)], xraw_v.at[pl.ds(0, xc)])

        for u2 in range(bw // 2):
            xw = xraw_v[pl.ds(2 * u2 * tp, _L)]
            for s, m in ((2 * u2, m_lo), (2 * u2 + 1, m_hi)):
                xv = jnp.sum(jnp.where(m, xw, zero), dtype=jnp.int32)
                base_s[s] = xv & ~(_W - 1)

        def fire(p, buf, sem):
            for u in range(_PIECE):
                a = pl.multiple_of(base_s[p * _PIECE + u], _W)
                pltpu.async_copy(
                    tblT_hbm.at[:, pl.ds(a, _W)],
                    win_v.at[np.int32(buf), :, pl.ds(u * _W, _W)],
                    sem,
                )

        def drain(buf, sem):
            pltpu.make_async_copy(
                tblT_hbm.at[:, pl.ds(0, _PIECE * _W)],
                win_v.at[np.int32(buf)],
                sem,
            ).wait()

        def extract(p, buf):
            bufv = zero + buf
            jb = p * _PIECE
            xvv = plsc.load_gather(xraw_v, [(lanes + jb) * tp + i])
            col = lanes * _W + (xvv & (_W - 1))
            base = (lanes + jb) * embed
            for e in range(embed):
                vals = plsc.load_gather(win_v, [bufv, zero + e, col])
                plsc.store_scatter(rv_v, [base + e], vals)

        for r in range(3):
            fire(r, r, sems[r])

        n_main = (n_p - 2) // 3

        def body(q, carry):
            for r in range(3):
                p = 3 * q + r
                drain(r, sems[r])
                extract(p, r)

                @pl.when(p + 3 < n_p)
                def _():
                    fire(p + 3, r, sems[r])

            return carry

        lax.fori_loop(
            jnp.int32(0), jnp.int32(n_main), body, jnp.int32(0), unroll=False
        )
        for p in range(3 * n_main, n_p):
            r = p % 3
            drain(r, sems[r])
            extract(p, r)

        pltpu.sync_copy(rv_v, out_hbm.at[i, pl.ds(j0 * embed, bw * embed)])

    return k


def kernel(x, table):
    vocab, embed = table.shape
    (batch,) = x.shape
    tp = 8
    xi = x.astype(jnp.int32)
    return _build_sc_gather(vocab, embed, batch, tp)(
        xi, table.astype(jnp.float32).T
    )
